# Initial kernel scaffold; baseline (speedup 1.0000x reference)
import functools

import jax
import jax.numpy as jnp
from jax import lax
from jax.experimental import pallas as pl
from jax.experimental.pallas import tpu as pltpu
from jax.experimental.pallas import tpu_sc as plsc

OUT_DIM = 32
BATCH = 16384
FIELDS = 26

B = BATCH * FIELDS
NC, NS = 2, 16
NW = NC * NS
B_PER_W = B // NW
CHUNK = 1024
N_CHUNKS = B_PER_W // CHUNK

_mesh = plsc.VectorSubcoreMesh(core_axis_name="c", subcore_axis_name="s")


@functools.partial(
    pl.kernel,
    mesh=_mesh,
    out_type=jax.ShapeDtypeStruct((B, OUT_DIM), jnp.float32),
    scratch_types=[
        pltpu.VMEM((CHUNK,), jnp.int32),
        pltpu.VMEM((CHUNK, OUT_DIM), jnp.float32),
        pltpu.SemaphoreType.DMA,
    ],
)
def _gather_kernel(idx_hbm, table_hbm, out_hbm, idx_v, rows_v, sem):
    wid = lax.axis_index("s") * NC + lax.axis_index("c")
    base = wid * B_PER_W

    def body(i, carry):
        off = base + i * CHUNK
        pltpu.sync_copy(idx_hbm.at[pl.ds(off, CHUNK)], idx_v)
        pltpu.async_copy(table_hbm.at[idx_v], rows_v, sem).wait()
        pltpu.sync_copy(rows_v, out_hbm.at[pl.ds(off, CHUNK)])
        return carry

    lax.fori_loop(0, N_CHUNKS, body, 0)


def kernel(in_tensor, table):
    idx = in_tensor.reshape(-1).astype(jnp.int32)
    out = _gather_kernel(idx, table)
    return out.reshape(BATCH, FIELDS, OUT_DIM)


# speedup vs baseline: 1.5540x; 1.5540x over previous
import functools

import jax
import jax.numpy as jnp
from jax import lax
from jax.experimental import pallas as pl
from jax.experimental.pallas import tpu as pltpu
from jax.experimental.pallas import tpu_sc as plsc

OUT_DIM = 32
BATCH = 16384
FIELDS = 26

B = BATCH * FIELDS
NC, NS = 2, 16
NW = NC * NS
B_PER_W = B // NW
CHUNK = 1024
N_CHUNKS = B_PER_W // CHUNK

_mesh = plsc.VectorSubcoreMesh(core_axis_name="c", subcore_axis_name="s")


@functools.partial(
    pl.kernel,
    mesh=_mesh,
    out_type=jax.ShapeDtypeStruct((B, OUT_DIM), jnp.float32),
    scratch_types=[
        pltpu.VMEM((CHUNK,), jnp.int32),
        pltpu.VMEM((CHUNK, OUT_DIM), jnp.float32),
        pltpu.SemaphoreType.DMA,
    ],
    compiler_params=pltpu.CompilerParams(use_tc_tiling_on_sc=False),
)
def _gather_kernel(idx_hbm, table_hbm, out_hbm, idx_v, rows_v, sem):
    wid = lax.axis_index("s") * NC + lax.axis_index("c")
    base = wid * B_PER_W

    def body(i, carry):
        off = base + i * CHUNK
        pltpu.sync_copy(idx_hbm.at[pl.ds(off, CHUNK)], idx_v)
        pltpu.async_copy(table_hbm.at[idx_v], rows_v, sem).wait()
        pltpu.sync_copy(rows_v, out_hbm.at[pl.ds(off, CHUNK)])
        return carry

    lax.fori_loop(0, N_CHUNKS, body, 0)


def kernel(in_tensor, table):
    idx = in_tensor.reshape(-1).astype(jnp.int32)
    out = _gather_kernel(idx, table)
    return out.reshape(BATCH, FIELDS, OUT_DIM)


# speedup vs baseline: 1.5762x; 1.0142x over previous
import functools

import jax
import jax.numpy as jnp
from jax import lax
from jax.experimental import pallas as pl
from jax.experimental.pallas import tpu as pltpu
from jax.experimental.pallas import tpu_sc as plsc

OUT_DIM = 32
BATCH = 16384
FIELDS = 26

B = BATCH * FIELDS
NC, NS = 2, 16
NW = NC * NS
B_PER_W = B // NW
CHUNK = 832
N_CHUNKS = B_PER_W // CHUNK
NBUF = 4

_mesh = plsc.VectorSubcoreMesh(core_axis_name="c", subcore_axis_name="s")


@functools.partial(
    pl.kernel,
    mesh=_mesh,
    out_type=jax.ShapeDtypeStruct((B, OUT_DIM), jnp.float32),
    scratch_types=[
        pltpu.VMEM((B_PER_W,), jnp.int32),
        pltpu.VMEM((NBUF, CHUNK, OUT_DIM), jnp.float32),
        [pltpu.SemaphoreType.DMA] * NBUF,
        [pltpu.SemaphoreType.DMA] * NBUF,
    ],
    compiler_params=pltpu.CompilerParams(use_tc_tiling_on_sc=False),
)
def _gather_kernel(idx_hbm, table_hbm, out_hbm, idx_v, rows_v, gsems, ssems):
    wid = lax.axis_index("s") * NC + lax.axis_index("c")
    base = wid * B_PER_W

    pltpu.sync_copy(idx_hbm.at[pl.ds(base, B_PER_W)], idx_v)

    gathers = [None] * N_CHUNKS
    stores = [None] * N_CHUNKS

    def start_gather(i):
        b = i % NBUF
        gathers[i] = pltpu.async_copy(
            table_hbm.at[idx_v.at[pl.ds(i * CHUNK, CHUNK)]],
            rows_v.at[b],
            gsems[b],
        )

    def start_store(i):
        b = i % NBUF
        stores[i] = pltpu.async_copy(
            rows_v.at[b],
            out_hbm.at[pl.ds(base + i * CHUNK, CHUNK)],
            ssems[b],
        )

    for i in range(NBUF - 1):
        start_gather(i)
    for i in range(N_CHUNKS):
        nxt = i + NBUF - 1
        if nxt < N_CHUNKS:
            if i >= 1:
                stores[i - 1].wait()
                stores[i - 1] = None
            start_gather(nxt)
        gathers[i].wait()
        start_store(i)
    for s in stores:
        if s is not None:
            s.wait()


def kernel(in_tensor, table):
    idx = in_tensor.reshape(-1).astype(jnp.int32)
    out = _gather_kernel(idx, table)
    return out.reshape(BATCH, FIELDS, OUT_DIM)
